# baseline (speedup 1.0000x reference)
import functools

import jax
import jax.numpy as jnp
from jax import lax
from jax.experimental import pallas as pl
from jax.experimental.pallas import tpu as pltpu
from jax.experimental.pallas import tpu_sc as plsc

B, NC, L = 8, 8, 2048
MAX_OBJS, MAX_EDGES, NFEAT, NID = 128, 8192, 128, 32
OUT_DIM = 256
NTOK = B * NC * L
CTAB_ROWS = 9216
NODE0 = MAX_EDGES
NODE2 = MAX_EDGES + 128

_PB = 1024


def _project_body(ef_ref, nf_ref, W_ref, b_ref, temb_lo_ref, out_ref):
    j = pl.program_id(1)
    f32 = jnp.float32
    eproj = jnp.dot(ef_ref[0], W_ref[...], preferred_element_type=f32)
    out_ref[0] = eproj + b_ref[...] + temb_lo_ref[1:2, :]

    @pl.when(j == 8)
    def _():
        nproj = jnp.dot(nf_ref[0], W_ref[...], preferred_element_type=f32)
        nproj = nproj + b_ref[...]
        out_ref[0, 0:128] = nproj + temb_lo_ref[0:1, :]
        out_ref[0, 128:256] = nproj + temb_lo_ref[2:3, :]


def _build_ctab(efeats_lup, nfeats_lup, attr_W, attr_b, temb_lo):
    return pl.pallas_call(
        _project_body,
        grid=(B, CTAB_ROWS // _PB),
        in_specs=[
            pl.BlockSpec((1, _PB, NFEAT),
                         lambda b, j: (b, jnp.minimum(j, 7), 0)),
            pl.BlockSpec((1, MAX_OBJS, NFEAT), lambda b, j: (b, 0, 0)),
            pl.BlockSpec((NFEAT, NFEAT), lambda b, j: (0, 0)),
            pl.BlockSpec((1, NFEAT), lambda b, j: (0, 0)),
            pl.BlockSpec((3, NFEAT), lambda b, j: (0, 0)),
        ],
        out_specs=pl.BlockSpec((1, _PB, NFEAT), lambda b, j: (b, j, 0)),
        out_shape=jax.ShapeDtypeStruct((B, CTAB_ROWS, NFEAT), jnp.float32),
    )(efeats_lup, nfeats_lup, attr_W, attr_b.reshape(1, -1), temb_lo)


_SC_WORKERS = 32
_TOK_PER_W = NTOK // _SC_WORKERS
_CHUNK = 128
_NCHUNK = _TOK_PER_W // _CHUNK


def _sc_gather(gidx2d, ctab_flat):
    mesh = plsc.VectorSubcoreMesh(core_axis_name="c", subcore_axis_name="s")

    @functools.partial(
        pl.kernel,
        out_type=jax.ShapeDtypeStruct((NTOK, 2, NFEAT), jnp.float32),
        mesh=mesh,
        scratch_types=[
            pltpu.VMEM((_NCHUNK, _CHUNK), jnp.int32),
            pltpu.VMEM((_CHUNK, NFEAT), jnp.float32),
            pltpu.SemaphoreType.DMA,
        ],
    )
    def k(gidx_hbm, tab_hbm, out_hbm, idx_v, rows_v, sem):
        wid = lax.axis_index("s") * 2 + lax.axis_index("c")
        base = wid * _TOK_PER_W
        pltpu.sync_copy(gidx_hbm.at[pl.ds(wid * _NCHUNK, _NCHUNK)], idx_v)

        def body(j, carry):
            pltpu.async_copy(tab_hbm.at[idx_v.at[j]], rows_v, sem).wait()
            pltpu.sync_copy(rows_v,
                            out_hbm.at[pl.ds(base + j * _CHUNK, _CHUNK), 0])
            return carry

        lax.fori_loop(0, _NCHUNK, body, 0)

    return k(gidx2d, ctab_flat)


_HI = lax.Precision.HIGHEST
_TWO_PI_HI = 6.28125
_TWO_PI_LO = 0.0019353071795864769
_INV_2PI = 0.15915494309189535


def _fast_cos(x):
    k = jnp.floor(x * _INV_2PI + 0.5)
    r = (x - k * _TWO_PI_HI) - k * _TWO_PI_LO
    r2 = r * r
    p = jnp.float32(1.7368827487e-09)
    p = p * r2 + jnp.float32(-2.7113293594e-07)
    p = p * r2 + jnp.float32(2.4773416502e-05)
    p = p * r2 + jnp.float32(-1.3887970073e-03)
    p = p * r2 + jnp.float32(4.1666524298e-02)
    p = p * r2 + jnp.float32(-4.9999991767e-01)
    p = p * r2 + jnp.float32(9.9999999227e-01)
    return p


def _tc_body(types_ref, time_ref, time2d_ref, bbox_ref, nidx_ref, nlup_ref,
             bbox_W_ref, bbox_b_ref, fmat_ref, ff_ref, pp_ref, nid_W_ref,
             nid_b_ref, temb_hi_ref, buf_ref, out_ref):
    del buf_ref
    f32 = jnp.float32
    LR = types_ref.shape[2]
    types = types_ref[0, 0]

    bbox = jnp.dot(bbox_ref[0, 0], bbox_W_ref[...], preferred_element_type=f32)
    bbox = bbox + bbox_b_ref[...]

    mx = jnp.max(time2d_ref[0, 0])
    base = mx * ff_ref[...] + pp_ref[...]
    tf = jnp.dot(time_ref[0, 0], fmat_ref[...],
                 preferred_element_type=f32, precision=_HI)
    h01 = _fast_cos(base - tf)

    ida = nidx_ref[0, 0][:, 0:1]
    idb = nidx_ref[0, 0][:, 1:2]
    iota_n = lax.broadcasted_iota(jnp.int32, (LR, NID), 1)
    oh_a = (ida == iota_n).astype(f32)
    oh_b = (idb == iota_n).astype(f32)
    p_top = jnp.dot(nlup_ref[0, 0], nid_W_ref[0:NID, :],
                    preferred_element_type=f32)
    p_bot = jnp.dot(nlup_ref[0, 0], nid_W_ref[NID:2 * NID, :],
                    preferred_element_type=f32)
    nid = (jnp.dot(oh_a, p_top, preferred_element_type=f32)
           + jnp.dot(oh_b, p_bot, preferred_element_type=f32)
           + nid_b_ref[...])

    iota_t = lax.broadcasted_iota(jnp.int32, (LR, 3), 1)
    oh_t = (types == iota_t).astype(f32)
    tfeat = jnp.dot(oh_t, temb_hi_ref[...], preferred_element_type=f32)

    out_ref[0, 0] = jnp.concatenate([bbox, h01, nid], axis=-1) + tfeat


def kernel(num_objs, token_pair_idx, token_pair_time, token_types, token_eidx,
           nfeats_lup, efeats_lup, bbox_feats, idx_in_lookup, n_id_lookup,
           attr_W, attr_b, bbox_W, bbox_b, time_freq, time_phase,
           n_id_W, n_id_b, type_emb):
    del num_objs
    temb_lo = type_emb[:, 0:NFEAT]
    temb_hi = type_emb[:, NFEAT:OUT_DIM]

    ctab = _build_ctab(efeats_lup, nfeats_lup, attr_W, attr_b, temb_lo)
    ctab_flat = ctab.reshape(B * CTAB_ROWS, NFEAT)

    tt = token_types.astype(jnp.int32)
    idx0 = token_pair_idx[..., 0].astype(jnp.int32)
    eidx = token_eidx.astype(jnp.int32)
    local = jnp.where(tt == 1, eidx,
                      NODE0 + jnp.where(tt == 2, 128, 0) + idx0)
    gidx = local + (jnp.arange(B, dtype=jnp.int32) * CTAB_ROWS)[:, None, None]
    gidx2d = gidx.reshape(NTOK // _CHUNK, _CHUNK)

    buf = _sc_gather(gidx2d, ctab_flat)
    buf4 = buf.reshape(B, NC, L, OUT_DIM)

    half = time_freq.shape[0]
    z = jnp.zeros((half,), jnp.float32)
    fmat = jnp.stack([jnp.concatenate([time_freq, z]),
                      jnp.concatenate([z, time_freq])])
    ff = jnp.concatenate([time_freq, time_freq])
    pp = jnp.concatenate([time_phase, time_phase])

    types_r = tt.reshape(B, NC, L, 1)
    nidx = idx_in_lookup.astype(jnp.int32).reshape(B, NC, L, 2)

    grid = (B, NC)
    bnc = lambda b, c: (b, c, 0, 0)
    full2 = lambda r, c: pl.BlockSpec((r, c), lambda b, n: (0, 0))

    out = pl.pallas_call(
        _tc_body,
        grid=grid,
        in_specs=[
            pl.BlockSpec((1, 1, L, 1), bnc),
            pl.BlockSpec((1, 1, L, 2), bnc),
            pl.BlockSpec((1, 1, 32, 128), bnc),
            pl.BlockSpec((1, 1, L, 8), bnc),
            pl.BlockSpec((1, 1, L, 2), bnc),
            pl.BlockSpec((1, 1, NID, NID), bnc),
            full2(8, 32),
            full2(1, 32),
            full2(2, 64),
            full2(1, 64),
            full2(1, 64),
            full2(2 * NID, 32),
            full2(1, 32),
            full2(3, NFEAT),
            pl.BlockSpec(memory_space=pltpu.MemorySpace.HBM),
        ],
        out_specs=pl.BlockSpec((1, 1, L, NFEAT), lambda b, n: (b, n, 0, 1)),
        out_shape=jax.ShapeDtypeStruct((B, NC, L, OUT_DIM), jnp.float32),
        input_output_aliases={14: 0},
    )(
        types_r, token_pair_time, token_pair_time.reshape(B, NC, 32, 128),
        bbox_feats, nidx, n_id_lookup, bbox_W, bbox_b.reshape(1, -1),
        fmat, ff.reshape(1, -1), pp.reshape(1, -1),
        n_id_W, n_id_b.reshape(1, -1), temb_hi, buf4,
    )
    return out


# speedup vs baseline: 2.0255x; 2.0255x over previous
import functools

import jax
import jax.numpy as jnp
from jax import lax
from jax.experimental import pallas as pl
from jax.experimental.pallas import tpu as pltpu
from jax.experimental.pallas import tpu_sc as plsc

B, NC, L = 8, 8, 2048
MAX_OBJS, MAX_EDGES, NFEAT, NID = 128, 8192, 128, 32
OUT_DIM = 256
NTOK = B * NC * L

_SC_WORKERS = 32
_TOK_PER_W = NTOK // _SC_WORKERS
_CHUNK = 128
_NCHUNK = _TOK_PER_W // _CHUNK


def _sc_gather(gidx2d, efeats_flat):
    mesh = plsc.VectorSubcoreMesh(core_axis_name="c", subcore_axis_name="s")

    @functools.partial(
        pl.kernel,
        out_type=jax.ShapeDtypeStruct((NTOK, NFEAT), jnp.float32),
        mesh=mesh,
        scratch_types=[
            pltpu.VMEM((_NCHUNK, _CHUNK), jnp.int32),
            pltpu.VMEM((_CHUNK, NFEAT), jnp.float32),
            pltpu.SemaphoreType.DMA,
        ],
    )
    def k(gidx_hbm, tab_hbm, out_hbm, idx_v, rows_v, sem):
        wid = lax.axis_index("s") * 2 + lax.axis_index("c")
        base = wid * _TOK_PER_W
        pltpu.sync_copy(gidx_hbm.at[pl.ds(wid * _NCHUNK, _NCHUNK)], idx_v)

        def body(j, carry):
            pltpu.async_copy(tab_hbm.at[idx_v.at[j]], rows_v, sem).wait()
            pltpu.sync_copy(rows_v, out_hbm.at[pl.ds(base + j * _CHUNK, _CHUNK)])
            return carry

        lax.fori_loop(0, _NCHUNK, body, 0)

    return k(gidx2d, efeats_flat)


_HI = lax.Precision.HIGHEST
_TWO_PI_HI = 6.28125
_TWO_PI_LO = 0.0019353071795864769
_INV_2PI = 0.15915494309189535


def _fast_cos(x):
    k = jnp.floor(x * _INV_2PI + 0.5)
    r = (x - k * _TWO_PI_HI) - k * _TWO_PI_LO
    r2 = r * r
    p = jnp.float32(1.7368827487e-09)
    p = p * r2 + jnp.float32(-2.7113293594e-07)
    p = p * r2 + jnp.float32(2.4773416502e-05)
    p = p * r2 + jnp.float32(-1.3887970073e-03)
    p = p * r2 + jnp.float32(4.1666524298e-02)
    p = p * r2 + jnp.float32(-4.9999991767e-01)
    p = p * r2 + jnp.float32(9.9999999227e-01)
    return p


def _dg0(a, b, precision=None):
    return lax.dot_general(a, b, (((0,), (0,)), ((), ())),
                           precision=precision,
                           preferred_element_type=jnp.float32)


def _tc_body(tpi_t_ref, pk_ref, time_t_ref, eg_ref, nf_ref, bbox_t_ref,
             nlup_ref, attr_W_ref, attr_b_ref, bbox_W_ref, bbox_b_ref,
             fmat_ref, ff_ref, pp_ref, nid_W_ref, nid_b_ref, temb_ref,
             emask_ref, out_ref):
    f32 = jnp.float32
    idx0_row = tpi_t_ref[0, 0][0:1, :]
    types_row = pk_ref[0, 0][0:1, :]
    ida_row = pk_ref[0, 0][1:2, :]
    idb_row = pk_ref[0, 0][2:3, :]

    iota3 = lax.broadcasted_iota(jnp.int32, (3, L), 0)
    oh_t_t = (iota3 == types_row).astype(f32)
    tfeat = _dg0(oh_t_t, temb_ref[...])
    em = _dg0(oh_t_t, emask_ref[...])

    iota_o = lax.broadcasted_iota(jnp.int32, (NFEAT, L), 0)
    oh_n_t = ((iota_o == idx0_row) & (types_row != 1)).astype(f32)
    nf = _dg0(oh_n_t, nf_ref[0])
    attr_feats = nf + eg_ref[0, 0] * em
    attr = jnp.dot(attr_feats, attr_W_ref[...], preferred_element_type=f32)
    attr = attr + attr_b_ref[...]

    bbox = _dg0(bbox_t_ref[0, 0], bbox_W_ref[...]) + bbox_b_ref[...]

    tt = time_t_ref[0, 0]
    mx = jnp.max(tt)
    base = mx * ff_ref[...] + pp_ref[...]
    tf = _dg0(tt, fmat_ref[...], precision=_HI)
    h01 = _fast_cos(base - tf)

    iota_n = lax.broadcasted_iota(jnp.int32, (NID, L), 0)
    oh_a_t = (iota_n == ida_row).astype(f32)
    oh_b_t = (iota_n == idb_row).astype(f32)
    p_top = jnp.dot(nlup_ref[0, 0], nid_W_ref[0:NID, :],
                    preferred_element_type=f32)
    p_bot = jnp.dot(nlup_ref[0, 0], nid_W_ref[NID:2 * NID, :],
                    preferred_element_type=f32)
    nid = _dg0(oh_a_t, p_top) + _dg0(oh_b_t, p_bot) + nid_b_ref[...]

    out_ref[0, 0] = jnp.concatenate([attr, bbox, h01, nid], axis=-1) + tfeat


def kernel(num_objs, token_pair_idx, token_pair_time, token_types, token_eidx,
           nfeats_lup, efeats_lup, bbox_feats, idx_in_lookup, n_id_lookup,
           attr_W, attr_b, bbox_W, bbox_b, time_freq, time_phase,
           n_id_W, n_id_b, type_emb):
    del num_objs
    gidx = (token_eidx.astype(jnp.int32)
            + (jnp.arange(B, dtype=jnp.int32) * MAX_EDGES)[:, None, None])
    gidx2d = gidx.reshape(NTOK // _CHUNK, _CHUNK)
    efeats_flat = efeats_lup.reshape(B * MAX_EDGES, NFEAT)

    egather = _sc_gather(gidx2d, efeats_flat).reshape(B, NC, L, NFEAT)

    tpi_t = jnp.transpose(token_pair_idx.astype(jnp.int32), (0, 1, 3, 2))
    time_t = jnp.transpose(token_pair_time, (0, 1, 3, 2))
    bbox_t = jnp.transpose(bbox_feats, (0, 1, 3, 2))

    nli = idx_in_lookup.astype(jnp.int32)
    pk = jnp.stack([token_types.astype(jnp.int32),
                    nli[:, :, 0::2], nli[:, :, 1::2]], axis=2)

    half = time_freq.shape[0]
    z = jnp.zeros((half,), jnp.float32)
    fmat = jnp.stack([jnp.concatenate([time_freq, z]),
                      jnp.concatenate([z, time_freq])])
    ff = jnp.concatenate([time_freq, time_freq])
    pp = jnp.concatenate([time_phase, time_phase])
    emask = jnp.zeros((3, NFEAT), jnp.float32).at[1].set(1.0)

    grid = (B, NC)
    bnc = lambda b, c: (b, c, 0, 0)
    full2 = lambda r, c: pl.BlockSpec((r, c), lambda b, n: (0, 0))

    out = pl.pallas_call(
        _tc_body,
        grid=grid,
        in_specs=[
            pl.BlockSpec((1, 1, 2, L), bnc),
            pl.BlockSpec((1, 1, 3, L), bnc),
            pl.BlockSpec((1, 1, 2, L), bnc),
            pl.BlockSpec((1, 1, L, NFEAT), bnc),
            pl.BlockSpec((1, MAX_OBJS, NFEAT), lambda b, n: (b, 0, 0)),
            pl.BlockSpec((1, 1, 8, L), bnc),
            pl.BlockSpec((1, 1, NID, NID), bnc),
            full2(NFEAT, 128),
            full2(1, 128),
            full2(8, 32),
            full2(1, 32),
            full2(2, 64),
            full2(1, 64),
            full2(1, 64),
            full2(2 * NID, 32),
            full2(1, 32),
            full2(3, OUT_DIM),
            full2(3, NFEAT),
        ],
        out_specs=pl.BlockSpec((1, 1, L, OUT_DIM), bnc),
        out_shape=jax.ShapeDtypeStruct((B, NC, L, OUT_DIM), jnp.float32),
    )(
        tpi_t, pk, time_t, egather, nfeats_lup, bbox_t, n_id_lookup,
        attr_W, attr_b.reshape(1, -1), bbox_W, bbox_b.reshape(1, -1),
        fmat, ff.reshape(1, -1), pp.reshape(1, -1),
        n_id_W, n_id_b.reshape(1, -1), type_emb, emask,
    )
    return out


# speedup vs baseline: 2.2247x; 1.0983x over previous
import functools

import jax
import jax.numpy as jnp
from jax import lax
from jax.experimental import pallas as pl
from jax.experimental.pallas import tpu as pltpu
from jax.experimental.pallas import tpu_sc as plsc

B, NC, L = 8, 8, 2048
MAX_OBJS, MAX_EDGES, NFEAT, NID = 128, 8192, 128, 32
OUT_DIM = 256
NTOK = B * NC * L
CROWS = MAX_EDGES + MAX_OBJS

_SC_WORKERS = 32
_TOK_PER_W = NTOK // _SC_WORKERS
_CHUNK = 128
_NCHUNK = _TOK_PER_W // _CHUNK


def _sc_gather(gidx2d, efeats_flat):
    mesh = plsc.VectorSubcoreMesh(core_axis_name="c", subcore_axis_name="s")

    @functools.partial(
        pl.kernel,
        out_type=jax.ShapeDtypeStruct((NTOK, NFEAT), jnp.float32),
        mesh=mesh,
        scratch_types=[
            pltpu.VMEM((_NCHUNK, _CHUNK), jnp.int32),
            pltpu.VMEM((_CHUNK, NFEAT), jnp.float32),
            pltpu.SemaphoreType.DMA,
        ],
    )
    def k(gidx_hbm, tab_hbm, out_hbm, idx_v, rows_v, sem):
        wid = lax.axis_index("s") * 2 + lax.axis_index("c")
        base = wid * _TOK_PER_W
        pltpu.sync_copy(gidx_hbm.at[pl.ds(wid * _NCHUNK, _NCHUNK)], idx_v)

        def body(j, carry):
            pltpu.async_copy(tab_hbm.at[idx_v.at[j]], rows_v, sem).wait()
            pltpu.sync_copy(rows_v, out_hbm.at[pl.ds(base + j * _CHUNK, _CHUNK)])
            return carry

        lax.fori_loop(0, _NCHUNK, body, 0)

    return k(gidx2d, efeats_flat)


_HI = lax.Precision.HIGHEST
_TWO_PI_HI = 6.28125
_TWO_PI_LO = 0.0019353071795864769
_INV_2PI = 0.15915494309189535


def _fast_cos(x):
    k = jnp.floor(x * _INV_2PI + 0.5)
    r = (x - k * _TWO_PI_HI) - k * _TWO_PI_LO
    r2 = r * r
    p = jnp.float32(1.7368827487e-09)
    p = p * r2 + jnp.float32(-2.7113293594e-07)
    p = p * r2 + jnp.float32(2.4773416502e-05)
    p = p * r2 + jnp.float32(-1.3887970073e-03)
    p = p * r2 + jnp.float32(4.1666524298e-02)
    p = p * r2 + jnp.float32(-4.9999991767e-01)
    p = p * r2 + jnp.float32(9.9999999227e-01)
    return p


def _dg0(a, b, precision=None):
    return lax.dot_general(a, b, (((0,), (0,)), ((), ())),
                           precision=precision,
                           preferred_element_type=jnp.float32)


def _tc_body(pk_ref, time_t_ref, eg_ref, bbox_t_ref,
             nlup_ref, attr_W_ref, attr_b_ref, bbox_W_ref, bbox_b_ref,
             fmat_ref, ff_ref, pp_ref, nid_W_ref, nid_b_ref, temb_ref,
             out_ref):
    f32 = jnp.float32
    types_row = pk_ref[0, 0][0:1, :]
    ida_row = pk_ref[0, 0][1:2, :]
    idb_row = pk_ref[0, 0][2:3, :]

    iota3 = lax.broadcasted_iota(jnp.int32, (3, L), 0)
    oh_t_t = (iota3 == types_row).astype(f32)
    tfeat = _dg0(oh_t_t, temb_ref[...])

    attr = jnp.dot(eg_ref[0, 0], attr_W_ref[...], preferred_element_type=f32)
    attr = attr + attr_b_ref[...]

    bbox = _dg0(bbox_t_ref[0, 0], bbox_W_ref[...]) + bbox_b_ref[...]

    tt = time_t_ref[0, 0]
    mx = jnp.max(tt)
    base = mx * ff_ref[...] + pp_ref[...]
    tf = _dg0(tt, fmat_ref[...], precision=_HI)
    h01 = _fast_cos(base - tf)

    iota_n = lax.broadcasted_iota(jnp.int32, (NID, L), 0)
    oh_a_t = (iota_n == ida_row).astype(f32)
    oh_b_t = (iota_n == idb_row).astype(f32)
    p_top = jnp.dot(nlup_ref[0, 0], nid_W_ref[0:NID, :],
                    preferred_element_type=f32)
    p_bot = jnp.dot(nlup_ref[0, 0], nid_W_ref[NID:2 * NID, :],
                    preferred_element_type=f32)
    nid = _dg0(oh_a_t, p_top) + _dg0(oh_b_t, p_bot) + nid_b_ref[...]

    out_ref[0, 0] = jnp.concatenate([attr, bbox, h01, nid], axis=-1) + tfeat


def kernel(num_objs, token_pair_idx, token_pair_time, token_types, token_eidx,
           nfeats_lup, efeats_lup, bbox_feats, idx_in_lookup, n_id_lookup,
           attr_W, attr_b, bbox_W, bbox_b, time_freq, time_phase,
           n_id_W, n_id_b, type_emb):
    del num_objs
    tt_i = token_types.astype(jnp.int32)
    local = jnp.where(tt_i == 1, token_eidx.astype(jnp.int32),
                      MAX_EDGES + token_pair_idx[..., 0].astype(jnp.int32))
    gidx = local + (jnp.arange(B, dtype=jnp.int32) * CROWS)[:, None, None]
    gidx2d = gidx.reshape(NTOK // _CHUNK, _CHUNK)
    ctab = jnp.concatenate([efeats_lup, nfeats_lup], axis=1)
    ctab_flat = ctab.reshape(B * CROWS, NFEAT)

    egather = _sc_gather(gidx2d, ctab_flat).reshape(B, NC, L, NFEAT)

    time_t = jnp.transpose(token_pair_time, (0, 1, 3, 2))
    bbox_t = jnp.transpose(bbox_feats, (0, 1, 3, 2))

    nli = idx_in_lookup.astype(jnp.int32)
    pk = jnp.stack([tt_i, nli[:, :, 0::2], nli[:, :, 1::2]], axis=2)

    half = time_freq.shape[0]
    z = jnp.zeros((half,), jnp.float32)
    fmat = jnp.stack([jnp.concatenate([time_freq, z]),
                      jnp.concatenate([z, time_freq])])
    ff = jnp.concatenate([time_freq, time_freq])
    pp = jnp.concatenate([time_phase, time_phase])
    grid = (B, NC)
    bnc = lambda b, c: (b, c, 0, 0)
    full2 = lambda r, c: pl.BlockSpec((r, c), lambda b, n: (0, 0))

    out = pl.pallas_call(
        _tc_body,
        grid=grid,
        in_specs=[
            pl.BlockSpec((1, 1, 3, L), bnc),
            pl.BlockSpec((1, 1, 2, L), bnc),
            pl.BlockSpec((1, 1, L, NFEAT), bnc),
            pl.BlockSpec((1, 1, 8, L), bnc),
            pl.BlockSpec((1, 1, NID, NID), bnc),
            full2(NFEAT, 128),
            full2(1, 128),
            full2(8, 32),
            full2(1, 32),
            full2(2, 64),
            full2(1, 64),
            full2(1, 64),
            full2(2 * NID, 32),
            full2(1, 32),
            full2(3, OUT_DIM),
        ],
        out_specs=pl.BlockSpec((1, 1, L, OUT_DIM), bnc),
        out_shape=jax.ShapeDtypeStruct((B, NC, L, OUT_DIM), jnp.float32),
    )(
        pk, time_t, egather, bbox_t, n_id_lookup,
        attr_W, attr_b.reshape(1, -1), bbox_W, bbox_b.reshape(1, -1),
        fmat, ff.reshape(1, -1), pp.reshape(1, -1),
        n_id_W, n_id_b.reshape(1, -1), type_emb,
    )
    return out


# speedup vs baseline: 2.3372x; 1.0506x over previous
import functools

import jax
import jax.numpy as jnp
from jax import lax
from jax.experimental import pallas as pl
from jax.experimental.pallas import tpu as pltpu
from jax.experimental.pallas import tpu_sc as plsc

B, NC, L = 8, 8, 2048
MAX_OBJS, MAX_EDGES, NFEAT, NID = 128, 8192, 128, 32
OUT_DIM = 256
NTOK = B * NC * L
CROWS = MAX_EDGES + MAX_OBJS

_SC_WORKERS = 32
_TOK_PER_W = NTOK // _SC_WORKERS
_CHUNK = 128
_NCHUNK = _TOK_PER_W // _CHUNK


def _sc_gather(gidx2d, efeats_flat):
    mesh = plsc.VectorSubcoreMesh(core_axis_name="c", subcore_axis_name="s")

    @functools.partial(
        pl.kernel,
        out_type=jax.ShapeDtypeStruct((NTOK, NFEAT), jnp.float32),
        mesh=mesh,
        scratch_types=[
            pltpu.VMEM((_NCHUNK, _CHUNK), jnp.int32),
            pltpu.VMEM((_CHUNK, NFEAT), jnp.float32),
            pltpu.SemaphoreType.DMA,
        ],
    )
    def k(gidx_hbm, tab_hbm, out_hbm, idx_v, rows_v, sem):
        wid = lax.axis_index("s") * 2 + lax.axis_index("c")
        base = wid * _TOK_PER_W
        pltpu.sync_copy(gidx_hbm.at[pl.ds(wid * _NCHUNK, _NCHUNK)], idx_v)

        def body(j, carry):
            pltpu.async_copy(tab_hbm.at[idx_v.at[j]], rows_v, sem).wait()
            pltpu.sync_copy(rows_v, out_hbm.at[pl.ds(base + j * _CHUNK, _CHUNK)])
            return carry

        lax.fori_loop(0, _NCHUNK, body, 0)

    return k(gidx2d, efeats_flat)


_HI = lax.Precision.HIGHEST
_TWO_PI_HI = 6.28125
_TWO_PI_LO = 0.0019353071795864769
_INV_2PI = 0.15915494309189535


def _fast_cos(x):
    k = jnp.floor(x * _INV_2PI + 0.5)
    r = (x - k * _TWO_PI_HI) - k * _TWO_PI_LO
    r2 = r * r
    p = jnp.float32(1.7368827487e-09)
    p = p * r2 + jnp.float32(-2.7113293594e-07)
    p = p * r2 + jnp.float32(2.4773416502e-05)
    p = p * r2 + jnp.float32(-1.3887970073e-03)
    p = p * r2 + jnp.float32(4.1666524298e-02)
    p = p * r2 + jnp.float32(-4.9999991767e-01)
    p = p * r2 + jnp.float32(9.9999999227e-01)
    return p


def _dg0(a, b, precision=None):
    return lax.dot_general(a, b, (((0,), (0,)), ((), ())),
                           precision=precision,
                           preferred_element_type=jnp.float32)


def _tc_body(pk_ref, time_t_ref, eg_ref, bbox_t_ref,
             nlup_ref, attr_W_ref, rhs12_ref,
             fmat_ref, ff_ref, pp_ref, nid_W_ref, temb_ref,
             out_ref):
    f32 = jnp.float32
    types_row = pk_ref[0, 0][0:1, :]
    ida_row = pk_ref[0, 0][1:2, :]
    idb_row = pk_ref[0, 0][2:3, :]

    iota3 = lax.broadcasted_iota(jnp.int32, (3, L), 0)
    oh_t_t = (iota3 == types_row).astype(f32)
    lhs12 = jnp.concatenate(
        [bbox_t_ref[0, 0], oh_t_t, jnp.ones((1, L), f32)], axis=0)
    bt = _dg0(lhs12, rhs12_ref[...])

    attr = jnp.dot(eg_ref[0, 0], attr_W_ref[...], preferred_element_type=f32)

    tt = time_t_ref[0, 0]
    mx = jnp.max(tt)
    base = mx * ff_ref[...] + pp_ref[...]
    tf = _dg0(tt, fmat_ref[...], precision=_HI)
    h01 = _fast_cos(base - tf)

    iota64 = lax.broadcasted_iota(jnp.int32, (2 * NID, L), 0)
    tgt = jnp.where(iota64 < NID, ida_row, idb_row)
    ohab = ((iota64 & (NID - 1)) == tgt).astype(f32)
    p_all = jnp.concatenate(
        [jnp.dot(nlup_ref[0, 0], nid_W_ref[0:NID, :],
                 preferred_element_type=f32),
         jnp.dot(nlup_ref[0, 0], nid_W_ref[NID:2 * NID, :],
                 preferred_element_type=f32)], axis=0)
    nid = _dg0(ohab, p_all)

    z32 = jnp.zeros((L, 32), f32)
    out_ref[0, 0] = jnp.concatenate([attr, z32, h01, nid], axis=-1) + bt


def kernel(num_objs, token_pair_idx, token_pair_time, token_types, token_eidx,
           nfeats_lup, efeats_lup, bbox_feats, idx_in_lookup, n_id_lookup,
           attr_W, attr_b, bbox_W, bbox_b, time_freq, time_phase,
           n_id_W, n_id_b, type_emb):
    del num_objs
    tt_i = token_types.astype(jnp.int32)
    local = jnp.where(tt_i == 1, token_eidx.astype(jnp.int32),
                      MAX_EDGES + token_pair_idx[..., 0].astype(jnp.int32))
    gidx = local + (jnp.arange(B, dtype=jnp.int32) * CROWS)[:, None, None]
    gidx2d = gidx.reshape(NTOK // _CHUNK, _CHUNK)
    ctab = jnp.concatenate([efeats_lup, nfeats_lup], axis=1)
    ctab_flat = ctab.reshape(B * CROWS, NFEAT)

    egather = _sc_gather(gidx2d, ctab_flat).reshape(B, NC, L, NFEAT)

    time_t = jnp.transpose(token_pair_time, (0, 1, 3, 2))
    bbox_t = jnp.transpose(bbox_feats, (0, 1, 3, 2))

    nli = idx_in_lookup.astype(jnp.int32)
    pk = jnp.stack([tt_i, nli[:, :, 0::2], nli[:, :, 1::2]], axis=2)

    half = time_freq.shape[0]
    z = jnp.zeros((half,), jnp.float32)
    fmat = jnp.stack([jnp.concatenate([time_freq, z]),
                      jnp.concatenate([z, time_freq])])
    ff = jnp.concatenate([time_freq, time_freq])
    pp = jnp.concatenate([time_phase, time_phase])
    bias256 = jnp.concatenate([attr_b, bbox_b, jnp.zeros((64,), jnp.float32),
                               n_id_b])
    rhs12 = jnp.zeros((12, OUT_DIM), jnp.float32)
    rhs12 = rhs12.at[0:8, 128:160].set(bbox_W)
    rhs12 = rhs12.at[8:11, :].set(type_emb)
    rhs12 = rhs12.at[11, :].set(bias256)
    grid = (B, NC)
    bnc = lambda b, c: (b, c, 0, 0)
    full2 = lambda r, c: pl.BlockSpec((r, c), lambda b, n: (0, 0))

    out = pl.pallas_call(
        _tc_body,
        grid=grid,
        in_specs=[
            pl.BlockSpec((1, 1, 3, L), bnc),
            pl.BlockSpec((1, 1, 2, L), bnc),
            pl.BlockSpec((1, 1, L, NFEAT), bnc),
            pl.BlockSpec((1, 1, 8, L), bnc),
            pl.BlockSpec((1, 1, NID, NID), bnc),
            full2(NFEAT, 128),
            full2(12, OUT_DIM),
            full2(2, 64),
            full2(1, 64),
            full2(1, 64),
            full2(2 * NID, 32),
            full2(3, OUT_DIM),
        ],
        out_specs=pl.BlockSpec((1, 1, L, OUT_DIM), bnc),
        out_shape=jax.ShapeDtypeStruct((B, NC, L, OUT_DIM), jnp.float32),
    )(
        pk, time_t, egather, bbox_t, n_id_lookup,
        attr_W, rhs12, fmat, ff.reshape(1, -1), pp.reshape(1, -1),
        n_id_W, type_emb,
    )
    return out


# speedup vs baseline: 2.4731x; 1.0582x over previous
import functools

import jax
import jax.numpy as jnp
from jax import lax
from jax.experimental import pallas as pl
from jax.experimental.pallas import tpu as pltpu
from jax.experimental.pallas import tpu_sc as plsc

B, NC, L = 8, 8, 2048
MAX_OBJS, MAX_EDGES, NFEAT, NID = 128, 8192, 128, 32
OUT_DIM = 256
NTOK = B * NC * L
CROWS = MAX_EDGES + MAX_OBJS

_SC_WORKERS = 32
_TOK_PER_W = NTOK // _SC_WORKERS
_CHUNK = 128
_NCHUNK = _TOK_PER_W // _CHUNK


def _sc_gather(gidx2d, efeats_flat):
    mesh = plsc.VectorSubcoreMesh(core_axis_name="c", subcore_axis_name="s")

    @functools.partial(
        pl.kernel,
        out_type=jax.ShapeDtypeStruct((NTOK, NFEAT), jnp.float32),
        mesh=mesh,
        scratch_types=[
            pltpu.VMEM((_NCHUNK, _CHUNK), jnp.int32),
            pltpu.VMEM((_CHUNK, NFEAT), jnp.float32),
            pltpu.VMEM((_CHUNK, NFEAT), jnp.float32),
            pltpu.SemaphoreType.DMA,
            pltpu.SemaphoreType.DMA,
        ],
    )
    def k(gidx_hbm, tab_hbm, out_hbm, idx_v, rows0, rows1, sem0, sem1):
        wid = lax.axis_index("s") * 2 + lax.axis_index("c")
        base = wid * _TOK_PER_W
        pltpu.sync_copy(gidx_hbm.at[pl.ds(wid * _NCHUNK, _NCHUNK)], idx_v)

        bufs = (rows0, rows1)
        sems = (sem0, sem1)
        pltpu.async_copy(tab_hbm.at[idx_v.at[0]], rows0, sem0)
        pltpu.async_copy(tab_hbm.at[idx_v.at[1]], rows1, sem1)

        def body(jj, carry):
            for bsel in (0, 1):
                j = 2 * jj + bsel
                buf, sem = bufs[bsel], sems[bsel]
                pltpu.make_async_copy(tab_hbm.at[idx_v.at[0]], buf, sem).wait()
                pltpu.sync_copy(
                    buf, out_hbm.at[pl.ds(base + j * _CHUNK, _CHUNK)])

                @pl.when(j + 2 < _NCHUNK)
                def _():
                    pltpu.async_copy(tab_hbm.at[idx_v.at[j + 2]], buf, sem)
            return carry

        lax.fori_loop(0, _NCHUNK // 2, body, 0)

    return k(gidx2d, efeats_flat)


_HI = lax.Precision.HIGHEST
_TWO_PI_HI = 6.28125
_TWO_PI_LO = 0.0019353071795864769
_INV_2PI = 0.15915494309189535


def _fast_cos(x):
    k = jnp.floor(x * _INV_2PI + 0.5)
    r = (x - k * _TWO_PI_HI) - k * _TWO_PI_LO
    r2 = r * r
    p = jnp.float32(1.7368827487e-09)
    p = p * r2 + jnp.float32(-2.7113293594e-07)
    p = p * r2 + jnp.float32(2.4773416502e-05)
    p = p * r2 + jnp.float32(-1.3887970073e-03)
    p = p * r2 + jnp.float32(4.1666524298e-02)
    p = p * r2 + jnp.float32(-4.9999991767e-01)
    p = p * r2 + jnp.float32(9.9999999227e-01)
    return p


def _dg0(a, b, precision=None):
    return lax.dot_general(a, b, (((0,), (0,)), ((), ())),
                           precision=precision,
                           preferred_element_type=jnp.float32)


def _tc_body(pk_ref, time_t_ref, eg_ref, bbox_t_ref,
             nlup_ref, attr_W_ref, rhs12_ref,
             fmat_ref, ff_ref, pp_ref, nid_W_ref, temb_ref,
             out_ref):
    f32 = jnp.float32
    types_row = pk_ref[0, 0][0:1, :]
    ida_row = pk_ref[0, 0][1:2, :]
    idb_row = pk_ref[0, 0][2:3, :]

    iota3 = lax.broadcasted_iota(jnp.int32, (3, L), 0)
    oh_t_t = (iota3 == types_row).astype(f32)
    lhs12 = jnp.concatenate(
        [bbox_t_ref[0, 0], oh_t_t, jnp.ones((1, L), f32)], axis=0)
    bt = _dg0(lhs12, rhs12_ref[...])

    attr = jnp.dot(eg_ref[0, 0], attr_W_ref[...], preferred_element_type=f32)

    tt = time_t_ref[0, 0]
    mx = jnp.max(tt)
    base = mx * ff_ref[...] + pp_ref[...]
    tf = _dg0(tt, fmat_ref[...], precision=_HI)
    h01 = _fast_cos(base - tf)

    iota64 = lax.broadcasted_iota(jnp.int32, (2 * NID, L), 0)
    tgt = jnp.where(iota64 < NID, ida_row, idb_row)
    ohab = ((iota64 & (NID - 1)) == tgt).astype(f32)
    p_all = jnp.concatenate(
        [jnp.dot(nlup_ref[0, 0], nid_W_ref[0:NID, :],
                 preferred_element_type=f32),
         jnp.dot(nlup_ref[0, 0], nid_W_ref[NID:2 * NID, :],
                 preferred_element_type=f32)], axis=0)
    nid = _dg0(ohab, p_all)

    z32 = jnp.zeros((L, 32), f32)
    out_ref[0, 0] = jnp.concatenate([attr, z32, h01, nid], axis=-1) + bt


def kernel(num_objs, token_pair_idx, token_pair_time, token_types, token_eidx,
           nfeats_lup, efeats_lup, bbox_feats, idx_in_lookup, n_id_lookup,
           attr_W, attr_b, bbox_W, bbox_b, time_freq, time_phase,
           n_id_W, n_id_b, type_emb):
    del num_objs
    tt_i = token_types.astype(jnp.int32)
    local = jnp.where(tt_i == 1, token_eidx.astype(jnp.int32),
                      MAX_EDGES + token_pair_idx[..., 0].astype(jnp.int32))
    gidx = local + (jnp.arange(B, dtype=jnp.int32) * CROWS)[:, None, None]
    gidx2d = gidx.reshape(NTOK // _CHUNK, _CHUNK)
    ctab = jnp.concatenate([efeats_lup, nfeats_lup], axis=1)
    ctab_flat = ctab.reshape(B * CROWS, NFEAT)

    egather = _sc_gather(gidx2d, ctab_flat).reshape(B, NC, L, NFEAT)

    time_t = jnp.transpose(token_pair_time, (0, 1, 3, 2))
    bbox_t = jnp.transpose(bbox_feats, (0, 1, 3, 2))

    nli = idx_in_lookup.astype(jnp.int32)
    pk = jnp.stack([tt_i, nli[:, :, 0::2], nli[:, :, 1::2]], axis=2)

    half = time_freq.shape[0]
    z = jnp.zeros((half,), jnp.float32)
    fmat = jnp.stack([jnp.concatenate([time_freq, z]),
                      jnp.concatenate([z, time_freq])])
    ff = jnp.concatenate([time_freq, time_freq])
    pp = jnp.concatenate([time_phase, time_phase])
    bias256 = jnp.concatenate([attr_b, bbox_b, jnp.zeros((64,), jnp.float32),
                               n_id_b])
    rhs12 = jnp.zeros((12, OUT_DIM), jnp.float32)
    rhs12 = rhs12.at[0:8, 128:160].set(bbox_W)
    rhs12 = rhs12.at[8:11, :].set(type_emb)
    rhs12 = rhs12.at[11, :].set(bias256)
    grid = (B, NC)
    bnc = lambda b, c: (b, c, 0, 0)
    full2 = lambda r, c: pl.BlockSpec((r, c), lambda b, n: (0, 0))

    out = pl.pallas_call(
        _tc_body,
        grid=grid,
        in_specs=[
            pl.BlockSpec((1, 1, 3, L), bnc),
            pl.BlockSpec((1, 1, 2, L), bnc),
            pl.BlockSpec((1, 1, L, NFEAT), bnc),
            pl.BlockSpec((1, 1, 8, L), bnc),
            pl.BlockSpec((1, 1, NID, NID), bnc),
            full2(NFEAT, 128),
            full2(12, OUT_DIM),
            full2(2, 64),
            full2(1, 64),
            full2(1, 64),
            full2(2 * NID, 32),
            full2(3, OUT_DIM),
        ],
        out_specs=pl.BlockSpec((1, 1, L, OUT_DIM), bnc),
        out_shape=jax.ShapeDtypeStruct((B, NC, L, OUT_DIM), jnp.float32),
    )(
        pk, time_t, egather, bbox_t, n_id_lookup,
        attr_W, rhs12, fmat, ff.reshape(1, -1), pp.reshape(1, -1),
        n_id_W, type_emb,
    )
    return out
